# baseline (speedup 1.0000x reference)
import functools

import jax
import jax.numpy as jnp
from jax import lax
from jax.experimental import pallas as pl
from jax.experimental.pallas import tpu as pltpu, tpu_sc as plsc

_BATCH = 16384
_SEQ = 50
_D = 16
_VOCAB = 1000000
_NW = 32
_ROWS_PER_W = _BATCH // _NW
_IDX_PER_W = _ROWS_PER_W * _SEQ
_IDX_COLS = 128
_IDX_ROWS = _IDX_PER_W // _IDX_COLS
_IDX2_ROWS = _BATCH * _SEQ // _IDX_COLS

_PROJ_BLK = 8192
_PROJ_GRID = (_VOCAB + _PROJ_BLK - 1) // _PROJ_BLK


def _proj_body(x_ref, w_ref, o_ref):
    y = lax.dot_general(x_ref[...], w_ref[...], (((1,), (0,)), ((), ())),
                        preferred_element_type=jnp.float32)
    o_ref[...] = y[:, 0]


def _sc_body(text_hbm, proj_hbm, b_hbm, out_hbm,
             idx_v, vals_v, b_v, out_v, sem):
    cid = lax.axis_index("c")
    sid = lax.axis_index("s")
    wid = cid * 16 + sid

    pltpu.sync_copy(b_hbm, b_v)
    pltpu.sync_copy(
        text_hbm.at[pl.ds(wid * _IDX_ROWS, _IDX_ROWS), :], idx_v)
    bv = b_v[...]
    lanes = lax.iota(jnp.int32, 16)

    for j in range(_IDX_ROWS):
        pltpu.async_copy(
            proj_hbm.at[idx_v.at[j]],
            vals_v.at[pl.ds(j * _IDX_COLS, _IDX_COLS)],
            sem)
    pltpu.make_async_copy(
        proj_hbm.at[pl.ds(0, _IDX_PER_W)], vals_v, sem).wait()

    def q_body(q, carry_q):
        row16 = q * 16 + lanes

        def l_body(l, acc):
            return acc + plsc.load_gather(vals_v, [row16 * _SEQ + l])

        acc = lax.fori_loop(0, _SEQ, l_body, jnp.zeros((16,), jnp.float32))
        out_v[pl.ds(q * 16, 16)] = acc + bv
        return carry_q

    lax.fori_loop(0, _ROWS_PER_W // 16, q_body, 0)
    pltpu.sync_copy(out_v, out_hbm.at[pl.ds(wid * _ROWS_PER_W, _ROWS_PER_W)])


@jax.jit
def _run(text2, embedding, w_col, b_vec):
    proj_flat = pl.pallas_call(
        _proj_body,
        grid=(_PROJ_GRID,),
        in_specs=[
            pl.BlockSpec((_PROJ_BLK, _D), lambda i: (i, 0)),
            pl.BlockSpec((_D, 1), lambda i: (0, 0)),
        ],
        out_specs=pl.BlockSpec((_PROJ_BLK,), lambda i: (i,)),
        out_shape=jax.ShapeDtypeStruct((_VOCAB,), jnp.float32),
    )(embedding, w_col)

    mesh = plsc.VectorSubcoreMesh(core_axis_name="c", subcore_axis_name="s")
    k = pl.kernel(
        _sc_body,
        out_type=jax.ShapeDtypeStruct((_BATCH,), jnp.float32),
        mesh=mesh,
        scratch_types=[
            pltpu.VMEM((_IDX_ROWS, _IDX_COLS), jnp.int32),
            pltpu.VMEM((_IDX_PER_W,), jnp.float32),
            pltpu.VMEM((16,), jnp.float32),
            pltpu.VMEM((_ROWS_PER_W,), jnp.float32),
            pltpu.SemaphoreType.DMA,
        ],
        compiler_params=pltpu.CompilerParams(
            use_tc_tiling_on_sc=False, needs_layout_passes=False),
    )
    return k(text2, proj_flat, b_vec)


def kernel(text, embedding, fc_w, fc_b):
    text2 = text.astype(jnp.int32).reshape(_IDX2_ROWS, _IDX_COLS)
    w_col = (fc_w[0] * (1.0 / _SEQ)).astype(jnp.float32)[:, None]
    b_vec = jnp.broadcast_to(fc_b.astype(jnp.float32), (16,))
    return _run(text2, embedding, w_col, b_vec)


# speedup vs baseline: 7.3247x; 7.3247x over previous
import functools

import jax
import jax.numpy as jnp
from jax import lax
from jax.experimental import pallas as pl
from jax.experimental.pallas import tpu as pltpu, tpu_sc as plsc

_BATCH = 16384
_SEQ = 50
_D = 16
_VOCAB = 1000000
_NW = 32
_ROWS_PER_W = _BATCH // _NW
_IDX_PER_W = _ROWS_PER_W * _SEQ
_IDX_COLS = 128
_IDX_ROWS = _IDX_PER_W // _IDX_COLS
_IDX2_ROWS = _BATCH * _SEQ // _IDX_COLS

_PROJ_BLK = 65536
_PROJ_GRID = (_VOCAB + _PROJ_BLK - 1) // _PROJ_BLK


def _proj_body(x_ref, w_ref, o_ref):
    o_ref[...] = jnp.sum(x_ref[...] * w_ref[...], axis=0)


def _sc_body(text_hbm, proj_hbm, b_hbm, out_hbm,
             idx_v, vals_v, b_v, out_v, sem):
    cid = lax.axis_index("c")
    sid = lax.axis_index("s")
    wid = cid * 16 + sid

    pltpu.sync_copy(b_hbm, b_v)
    pltpu.sync_copy(
        text_hbm.at[pl.ds(wid * _IDX_ROWS, _IDX_ROWS), :], idx_v)
    bv = b_v[...]
    lanes = lax.iota(jnp.int32, 16)

    for j in range(_IDX_ROWS):
        pltpu.async_copy(
            proj_hbm.at[idx_v.at[j]],
            vals_v.at[pl.ds(j * _IDX_COLS, _IDX_COLS)],
            sem)
    pltpu.make_async_copy(
        proj_hbm.at[pl.ds(0, _IDX_PER_W)], vals_v, sem).wait()

    def q_body(q, carry_q):
        row16 = q * 16 + lanes

        def l_body(l, acc):
            return acc + plsc.load_gather(vals_v, [row16 * _SEQ + l])

        acc = lax.fori_loop(0, _SEQ, l_body, jnp.zeros((16,), jnp.float32))
        out_v[pl.ds(q * 16, 16)] = acc + bv
        return carry_q

    lax.fori_loop(0, _ROWS_PER_W // 16, q_body, 0)
    pltpu.sync_copy(out_v, out_hbm.at[pl.ds(wid * _ROWS_PER_W, _ROWS_PER_W)])


@jax.jit
def _run(text2, emb_t, w_col, b_vec):
    proj_flat = pl.pallas_call(
        _proj_body,
        grid=(_PROJ_GRID,),
        in_specs=[
            pl.BlockSpec((_D, _PROJ_BLK), lambda i: (0, i)),
            pl.BlockSpec((_D, 1), lambda i: (0, 0)),
        ],
        out_specs=pl.BlockSpec((_PROJ_BLK,), lambda i: (i,)),
        out_shape=jax.ShapeDtypeStruct((_VOCAB,), jnp.float32),
    )(emb_t, w_col)

    mesh = plsc.VectorSubcoreMesh(core_axis_name="c", subcore_axis_name="s")
    k = pl.kernel(
        _sc_body,
        out_type=jax.ShapeDtypeStruct((_BATCH,), jnp.float32),
        mesh=mesh,
        scratch_types=[
            pltpu.VMEM((_IDX_ROWS, _IDX_COLS), jnp.int32),
            pltpu.VMEM((_IDX_PER_W,), jnp.float32),
            pltpu.VMEM((16,), jnp.float32),
            pltpu.VMEM((_ROWS_PER_W,), jnp.float32),
            pltpu.SemaphoreType.DMA,
        ],
        compiler_params=pltpu.CompilerParams(
            use_tc_tiling_on_sc=False, needs_layout_passes=False),
    )
    return k(text2, proj_flat, b_vec)


def kernel(text, embedding, fc_w, fc_b):
    text2 = text.astype(jnp.int32).reshape(_IDX2_ROWS, _IDX_COLS)
    w_col = (fc_w[0] * (1.0 / _SEQ)).astype(jnp.float32)[:, None]
    b_vec = jnp.broadcast_to(fc_b.astype(jnp.float32), (16,))
    return _run(text2, embedding.T, w_col, b_vec)


# speedup vs baseline: 8.7330x; 1.1923x over previous
import functools

import jax
import jax.numpy as jnp
from jax import lax
from jax.experimental import pallas as pl
from jax.experimental.pallas import tpu as pltpu, tpu_sc as plsc

_BATCH = 16384
_SEQ = 50
_D = 16
_VOCAB = 1000000
_NW = 32
_ROWS_PER_W = _BATCH // _NW
_IDX_PER_W = _ROWS_PER_W * _SEQ
_IDX_COLS = 128
_IDX_ROWS = _IDX_PER_W // _IDX_COLS
_IDX2_ROWS = _BATCH * _SEQ // _IDX_COLS

_PROJ_BLK = 65536
_PROJ_GRID = (_VOCAB + _PROJ_BLK - 1) // _PROJ_BLK


def _proj_body(x_ref, w_ref, o_ref):
    o_ref[...] = jnp.sum(x_ref[...] * w_ref[...], axis=0)


def _sc_body(text_hbm, proj_hbm, b_hbm, out_hbm,
             idx_v, vals_v, b_v, out_v, sem):
    cid = lax.axis_index("c")
    sid = lax.axis_index("s")
    wid = cid * 16 + sid

    pltpu.sync_copy(b_hbm, b_v)
    pltpu.sync_copy(
        text_hbm.at[:, pl.ds(wid * _ROWS_PER_W, _ROWS_PER_W)], idx_v)
    bv = b_v[...]

    for l in range(_SEQ):
        for c in range(_ROWS_PER_W // _IDX_COLS):
            pltpu.async_copy(
                proj_hbm.at[idx_v.at[l, pl.ds(c * _IDX_COLS, _IDX_COLS)]],
                vals_v.at[l, pl.ds(c * _IDX_COLS, _IDX_COLS)],
                sem)
    for l in range(_SEQ):
        pltpu.make_async_copy(
            proj_hbm.at[pl.ds(0, _ROWS_PER_W)], vals_v.at[l], sem).wait()

    def q_body(q, carry_q):
        col = q * 16
        acc = bv
        for l in range(_SEQ):
            acc = acc + vals_v[l, pl.ds(col, 16)]
        out_v[pl.ds(col, 16)] = acc
        return carry_q

    lax.fori_loop(0, _ROWS_PER_W // 16, q_body, 0)
    pltpu.sync_copy(out_v, out_hbm.at[pl.ds(wid * _ROWS_PER_W, _ROWS_PER_W)])


@jax.jit
def _run(text2, emb_t, w_col, b_vec):
    proj_flat = pl.pallas_call(
        _proj_body,
        grid=(_PROJ_GRID,),
        in_specs=[
            pl.BlockSpec((_D, _PROJ_BLK), lambda i: (0, i)),
            pl.BlockSpec((_D, 1), lambda i: (0, 0)),
        ],
        out_specs=pl.BlockSpec((_PROJ_BLK,), lambda i: (i,)),
        out_shape=jax.ShapeDtypeStruct((_VOCAB,), jnp.float32),
    )(emb_t, w_col)

    mesh = plsc.VectorSubcoreMesh(core_axis_name="c", subcore_axis_name="s")
    k = pl.kernel(
        _sc_body,
        out_type=jax.ShapeDtypeStruct((_BATCH,), jnp.float32),
        mesh=mesh,
        scratch_types=[
            pltpu.VMEM((_SEQ, _ROWS_PER_W), jnp.int32),
            pltpu.VMEM((_SEQ, _ROWS_PER_W), jnp.float32),
            pltpu.VMEM((16,), jnp.float32),
            pltpu.VMEM((_ROWS_PER_W,), jnp.float32),
            pltpu.SemaphoreType.DMA,
        ],
        compiler_params=pltpu.CompilerParams(
            use_tc_tiling_on_sc=False, needs_layout_passes=False),
    )
    return k(text2, proj_flat, b_vec)


def kernel(text, embedding, fc_w, fc_b):
    text_t = text.astype(jnp.int32).T
    w_col = (fc_w[0] * (1.0 / _SEQ)).astype(jnp.float32)[:, None]
    b_vec = jnp.broadcast_to(fc_b.astype(jnp.float32), (16,))
    return _run(text_t, embedding.T, w_col, b_vec)


# speedup vs baseline: 9.0414x; 1.0353x over previous
import functools

import jax
import jax.numpy as jnp
from jax import lax
from jax.experimental import pallas as pl
from jax.experimental.pallas import tpu as pltpu, tpu_sc as plsc

_BATCH = 16384
_SEQ = 50
_D = 16
_VOCAB = 1000000
_NW = 32
_ROWS_PER_W = _BATCH // _NW
_IDX_PER_W = _ROWS_PER_W * _SEQ
_IDX_COLS = 128
_IDX_ROWS = _IDX_PER_W // _IDX_COLS
_IDX2_ROWS = _BATCH * _SEQ // _IDX_COLS

_PROJ_BLK = 65536
_PROJ_GRID = (_VOCAB + _PROJ_BLK - 1) // _PROJ_BLK


def _proj_body(x_ref, w_ref, o_ref):
    y = lax.dot_general(w_ref[...], x_ref[...], (((0,), (0,)), ((), ())),
                        preferred_element_type=jnp.float32)
    o_ref[...] = y[0]


def _sc_body(text_hbm, proj_hbm, b_hbm, out_hbm,
             idx_v, vals_v, b_v, out_v, sem):
    cid = lax.axis_index("c")
    sid = lax.axis_index("s")
    wid = cid * 16 + sid

    pltpu.sync_copy(b_hbm, b_v)
    pltpu.sync_copy(
        text_hbm.at[:, pl.ds(wid * _ROWS_PER_W, _ROWS_PER_W)], idx_v)
    bv = b_v[...]

    for l in range(_SEQ):
        for c in range(_ROWS_PER_W // _IDX_COLS):
            pltpu.async_copy(
                proj_hbm.at[idx_v.at[l, pl.ds(c * _IDX_COLS, _IDX_COLS)]],
                vals_v.at[l, pl.ds(c * _IDX_COLS, _IDX_COLS)],
                sem)
    for l in range(_SEQ):
        pltpu.make_async_copy(
            proj_hbm.at[pl.ds(0, _ROWS_PER_W)], vals_v.at[l], sem).wait()

    def q_body(q, carry_q):
        col = q * 16
        acc = bv
        for l in range(_SEQ):
            acc = acc + vals_v[l, pl.ds(col, 16)]
        out_v[pl.ds(col, 16)] = acc
        return carry_q

    lax.fori_loop(0, _ROWS_PER_W // 16, q_body, 0)
    pltpu.sync_copy(out_v, out_hbm.at[pl.ds(wid * _ROWS_PER_W, _ROWS_PER_W)])


@jax.jit
def _run(text2, emb_t, w_col, b_vec):
    proj_flat = pl.pallas_call(
        _proj_body,
        grid=(_PROJ_GRID,),
        in_specs=[
            pl.BlockSpec((_D, _PROJ_BLK), lambda i: (0, i)),
            pl.BlockSpec((_D, 1), lambda i: (0, 0)),
        ],
        out_specs=pl.BlockSpec((_PROJ_BLK,), lambda i: (i,)),
        out_shape=jax.ShapeDtypeStruct((_VOCAB,), jnp.float32),
    )(emb_t, w_col)

    mesh = plsc.VectorSubcoreMesh(core_axis_name="c", subcore_axis_name="s")
    k = pl.kernel(
        _sc_body,
        out_type=jax.ShapeDtypeStruct((_BATCH,), jnp.float32),
        mesh=mesh,
        scratch_types=[
            pltpu.VMEM((_SEQ, _ROWS_PER_W), jnp.int32),
            pltpu.VMEM((_SEQ, _ROWS_PER_W), jnp.float32),
            pltpu.VMEM((16,), jnp.float32),
            pltpu.VMEM((_ROWS_PER_W,), jnp.float32),
            pltpu.SemaphoreType.DMA,
        ],
        compiler_params=pltpu.CompilerParams(
            use_tc_tiling_on_sc=False, needs_layout_passes=False),
    )
    return k(text2, proj_flat, b_vec)


def kernel(text, embedding, fc_w, fc_b):
    text_t = text.astype(jnp.int32).T
    w_col = (fc_w[0] * (1.0 / _SEQ)).astype(jnp.float32)[:, None]
    b_vec = jnp.broadcast_to(fc_b.astype(jnp.float32), (16,))
    return _run(text_t, embedding.T, w_col, b_vec)


# speedup vs baseline: 9.3263x; 1.0315x over previous
import functools

import jax
import jax.numpy as jnp
from jax import lax
from jax.experimental import pallas as pl
from jax.experimental.pallas import tpu as pltpu, tpu_sc as plsc

_BATCH = 16384
_SEQ = 50
_D = 16
_VOCAB = 1000000
_NW = 32
_ROWS_PER_W = _BATCH // _NW
_IDX_PER_W = _ROWS_PER_W * _SEQ
_IDX_COLS = 128
_IDX_ROWS = _IDX_PER_W // _IDX_COLS
_IDX2_ROWS = _BATCH * _SEQ // _IDX_COLS

_PROJ_BLK = 131072
_PROJ_GRID = (_VOCAB + _PROJ_BLK - 1) // _PROJ_BLK


def _proj_body(x_ref, w_ref, o_ref):
    y = lax.dot_general(w_ref[...], x_ref[...], (((0,), (0,)), ((), ())),
                        preferred_element_type=jnp.float32)
    o_ref[...] = y[0]


def _sc_body(text_hbm, proj_hbm, b_hbm, out_hbm,
             idx_v, vals_v, b_v, out_v, sem):
    cid = lax.axis_index("c")
    sid = lax.axis_index("s")
    wid = cid * 16 + sid

    pltpu.sync_copy(b_hbm, b_v)
    pltpu.sync_copy(
        text_hbm.at[:, pl.ds(wid * _ROWS_PER_W, _ROWS_PER_W)], idx_v)
    bv = b_v[...]

    for l in range(_SEQ):
        for c in range(_ROWS_PER_W // _IDX_COLS):
            pltpu.async_copy(
                proj_hbm.at[idx_v.at[l, pl.ds(c * _IDX_COLS, _IDX_COLS)]],
                vals_v.at[l, pl.ds(c * _IDX_COLS, _IDX_COLS)],
                sem)
    for l in range(_SEQ):
        pltpu.make_async_copy(
            proj_hbm.at[pl.ds(0, _ROWS_PER_W)], vals_v.at[l], sem).wait()

    def q_body(q, carry_q):
        col = q * 16
        acc = bv
        for l in range(_SEQ):
            acc = acc + vals_v[l, pl.ds(col, 16)]
        out_v[pl.ds(col, 16)] = acc
        return carry_q

    lax.fori_loop(0, _ROWS_PER_W // 16, q_body, 0)
    pltpu.sync_copy(out_v, out_hbm.at[pl.ds(wid * _ROWS_PER_W, _ROWS_PER_W)])


@jax.jit
def _run(text2, emb_t, w_col, b_vec):
    proj_flat = pl.pallas_call(
        _proj_body,
        grid=(_PROJ_GRID,),
        in_specs=[
            pl.BlockSpec((_D, _PROJ_BLK), lambda i: (0, i)),
            pl.BlockSpec((_D, 1), lambda i: (0, 0)),
        ],
        out_specs=pl.BlockSpec((_PROJ_BLK,), lambda i: (i,)),
        out_shape=jax.ShapeDtypeStruct((_VOCAB,), jnp.float32),
    )(emb_t, w_col)

    mesh = plsc.VectorSubcoreMesh(core_axis_name="c", subcore_axis_name="s")
    k = pl.kernel(
        _sc_body,
        out_type=jax.ShapeDtypeStruct((_BATCH,), jnp.float32),
        mesh=mesh,
        scratch_types=[
            pltpu.VMEM((_SEQ, _ROWS_PER_W), jnp.int32),
            pltpu.VMEM((_SEQ, _ROWS_PER_W), jnp.float32),
            pltpu.VMEM((16,), jnp.float32),
            pltpu.VMEM((_ROWS_PER_W,), jnp.float32),
            pltpu.SemaphoreType.DMA,
        ],
        compiler_params=pltpu.CompilerParams(
            use_tc_tiling_on_sc=False, needs_layout_passes=False),
    )
    return k(text2, proj_flat, b_vec)


def kernel(text, embedding, fc_w, fc_b):
    text_t = text.astype(jnp.int32).T
    w_col = (fc_w[0] * (1.0 / _SEQ)).astype(jnp.float32)[:, None]
    b_vec = jnp.broadcast_to(fc_b.astype(jnp.float32), (16,))
    return _run(text_t, embedding.T, w_col, b_vec)


# speedup vs baseline: 9.4030x; 1.0082x over previous
import functools

import jax
import jax.numpy as jnp
from jax import lax
from jax.experimental import pallas as pl
from jax.experimental.pallas import tpu as pltpu, tpu_sc as plsc

_BATCH = 16384
_SEQ = 50
_D = 16
_VOCAB = 1000000
_NW = 32
_ROWS_PER_W = _BATCH // _NW
_IDX_PER_W = _ROWS_PER_W * _SEQ
_IDX_COLS = 128
_IDX_ROWS = _IDX_PER_W // _IDX_COLS
_IDX2_ROWS = _BATCH * _SEQ // _IDX_COLS

_PROJ_BLK = 131072
_PROJ_GRID = (_VOCAB + _PROJ_BLK - 1) // _PROJ_BLK


def _proj_body(x_ref, w_ref, o_ref):
    y = lax.dot_general(w_ref[...], x_ref[...], (((0,), (0,)), ((), ())),
                        preferred_element_type=jnp.float32)
    o_ref[...] = y[0]


def _sc_body(text_hbm, proj_hbm, b_hbm, out_hbm,
             idx_v, vals_v, b_v, out_v, sem):
    cid = lax.axis_index("c")
    sid = lax.axis_index("s")
    wid = cid * 16 + sid

    pltpu.sync_copy(b_hbm, b_v)
    pltpu.sync_copy(
        text_hbm.at[:, pl.ds(wid * _ROWS_PER_W, _ROWS_PER_W)], idx_v)
    bv = b_v[...]

    for l in range(_SEQ):
        pltpu.async_copy(
            proj_hbm.at[idx_v.at[l]],
            vals_v.at[l],
            sem)
    for l in range(_SEQ):
        pltpu.make_async_copy(
            proj_hbm.at[pl.ds(0, _ROWS_PER_W)], vals_v.at[l], sem).wait()

    def q_body(q, carry_q):
        col = q * 16
        acc = bv
        for l in range(_SEQ):
            acc = acc + vals_v[l, pl.ds(col, 16)]
        out_v[pl.ds(col, 16)] = acc
        return carry_q

    lax.fori_loop(0, _ROWS_PER_W // 16, q_body, 0)
    pltpu.sync_copy(out_v, out_hbm.at[pl.ds(wid * _ROWS_PER_W, _ROWS_PER_W)])


@jax.jit
def _run(text2, emb_t, w_col, b_vec):
    proj_flat = pl.pallas_call(
        _proj_body,
        grid=(_PROJ_GRID,),
        in_specs=[
            pl.BlockSpec((_D, _PROJ_BLK), lambda i: (0, i)),
            pl.BlockSpec((_D, 1), lambda i: (0, 0)),
        ],
        out_specs=pl.BlockSpec((_PROJ_BLK,), lambda i: (i,)),
        out_shape=jax.ShapeDtypeStruct((_VOCAB,), jnp.float32),
    )(emb_t, w_col)

    mesh = plsc.VectorSubcoreMesh(core_axis_name="c", subcore_axis_name="s")
    k = pl.kernel(
        _sc_body,
        out_type=jax.ShapeDtypeStruct((_BATCH,), jnp.float32),
        mesh=mesh,
        scratch_types=[
            pltpu.VMEM((_SEQ, _ROWS_PER_W), jnp.int32),
            pltpu.VMEM((_SEQ, _ROWS_PER_W), jnp.float32),
            pltpu.VMEM((16,), jnp.float32),
            pltpu.VMEM((_ROWS_PER_W,), jnp.float32),
            pltpu.SemaphoreType.DMA,
        ],
        compiler_params=pltpu.CompilerParams(
            use_tc_tiling_on_sc=False, needs_layout_passes=False),
    )
    return k(text2, proj_flat, b_vec)


def kernel(text, embedding, fc_w, fc_b):
    text_t = text.astype(jnp.int32).T
    w_col = (fc_w[0] * (1.0 / _SEQ)).astype(jnp.float32)[:, None]
    b_vec = jnp.broadcast_to(fc_b.astype(jnp.float32), (16,))
    return _run(text_t, embedding.T, w_col, b_vec)
